# baseline (speedup 1.0000x reference)
import functools
import math

import jax
import jax.numpy as jnp
from jax import lax
from jax.experimental import pallas as pl
from jax.experimental.pallas import tpu as pltpu
from jax.experimental.pallas import tpu_sc as plsc

_G = 32
_MAX_T = 10000
_STEP_T = 1
_T = min(1 + _STEP_T, _MAX_T)
_DENO = math.log(_G) / (0.9 * _MAX_T)
_KSIZE = _G * math.exp(-_T * _DENO)
_SIGMA = 0.3 * ((_KSIZE - 1) * 0.5 - 1) + 0.8
_TWO_SIGMA_SQ = 2.0 * _SIGMA**2
_TEMP = math.exp(-(_T * 2) / _MAX_T)
_LANES = 16


def _dist_body(x_ref, u_ref, dist_ref):
    d = u_ref.shape[1]
    xc = x_ref[:] - 0.5
    uc = u_ref[:] - 0.5
    dn = (((1,), (1,)), ((), ()))
    un = jax.lax.dot_general(
        jnp.ones((1, d), jnp.float32), uc * uc, dn,
        preferred_element_type=jnp.float32,
        precision=jax.lax.Precision.HIGHEST)
    dot = jax.lax.dot_general(
        xc, uc, dn, preferred_element_type=jnp.float32,
        precision=jax.lax.Precision.HIGHEST)
    xn = jnp.sum(xc * xc, axis=1, keepdims=True)
    dist_ref[:] = (xn - 2.0 * dot) + un


_GDN = lax.GatherDimensionNumbers(
    offset_dims=(), collapsed_slice_dims=(0,), start_index_map=(0,))


def _shuffle(v, idx):
    return lax.gather(
        v, idx[:, None], _GDN, (1,),
        mode=lax.GatherScatterMode.PROMISE_IN_BOUNDS)


def _hmin(v, lane):
    for sh in (8, 4, 2, 1):
        v = jnp.minimum(v, _shuffle(v, lane ^ sh))
    return v


def _sc_body(nc, spw, dist_hbm, bmu_hbm, part_hbm, dvec, bmu_buf, acc_buf):
    n = dist_hbm.shape[1]
    nchunk = n // _LANES
    wid = lax.axis_index("s") * nc + lax.axis_index("c")
    pltpu.sync_copy(dist_hbm.at[pl.ds(wid * spw, spw)], dvec)
    lane = lax.iota(jnp.int32, _LANES)
    acc = jnp.zeros((_LANES,), jnp.float32)
    bmu_vec = jnp.zeros((_LANES,), jnp.int32)
    for s in range(spw):
        def amin_body(c, carry):
            minv, argv = carry
            v = dvec[s, pl.ds(c * _LANES, _LANES)]
            col = c * _LANES + lane
            m = v < minv
            return jnp.where(m, v, minv), jnp.where(m, col, argv)

        minv, argv = lax.fori_loop(
            1, nchunk, amin_body,
            (dvec[s, pl.ds(0, _LANES)], lane))
        mval = _hmin(minv, lane)
        bidx = _hmin(jnp.where(minv == mval, argv, jnp.int32(1 << 30)), lane)
        bmu_vec = jnp.where(lane == s, bidx, bmu_vec)
        by = bidx >> 5
        bx = bidx & 31

        def loss_body(c, acc):
            v = dvec[s, pl.ds(c * _LANES, _LANES)]
            col = c * _LANES + lane
            dy = (col >> 5) - by
            dx = (col & 31) - bx
            pd = (dy * dy + dx * dx).astype(jnp.float32)
            g = jnp.exp(-pd / _TWO_SIGMA_SQ)
            g = jnp.where(g < 0.001, 0.0, g)
            return acc + g * g * v

        acc = lax.fori_loop(0, nchunk, loss_body, acc)
    bmu_buf[:] = bmu_vec
    acc_buf[:] = acc
    pltpu.sync_copy(bmu_buf, bmu_hbm.at[wid])
    pltpu.sync_copy(acc_buf, part_hbm.at[wid])


def kernel(x, units):
    bsz = x.shape[0]
    d = units.shape[-1]
    n = units.shape[0] * units.shape[1]
    x2 = x.reshape(bsz, d)
    u2 = units.reshape(n, d)
    dist = pl.pallas_call(
        _dist_body,
        out_shape=jax.ShapeDtypeStruct((bsz, n), jnp.float32),
    )(x2, u2)

    info = plsc.get_sparse_core_info()
    nw = info.num_cores * info.num_subcores
    spw = bsz // nw
    mesh = plsc.VectorSubcoreMesh(core_axis_name="c", subcore_axis_name="s")
    sc = pl.kernel(
        functools.partial(_sc_body, info.num_cores, spw),
        mesh=mesh,
        out_type=(
            jax.ShapeDtypeStruct((nw, _LANES), jnp.int32),
            jax.ShapeDtypeStruct((nw, _LANES), jnp.float32),
        ),
        scratch_types=[
            pltpu.VMEM((spw, n), jnp.float32),
            pltpu.VMEM((_LANES,), jnp.int32),
            pltpu.VMEM((_LANES,), jnp.float32),
        ],
    )
    bmu_parts, loss_parts = sc(dist)
    bmu = bmu_parts[:, :spw].reshape(bsz, 1)
    loss = jnp.sum(loss_parts) * (_TEMP * _TEMP / (bsz * n * d))
    return bmu, loss


# speedup vs baseline: 1.0010x; 1.0010x over previous
import functools
import math

import jax
import jax.numpy as jnp
from jax import lax
from jax.experimental import pallas as pl
from jax.experimental.pallas import tpu as pltpu
from jax.experimental.pallas import tpu_sc as plsc

_G = 32
_MAX_T = 10000
_STEP_T = 1
_T = min(1 + _STEP_T, _MAX_T)
_DENO = math.log(_G) / (0.9 * _MAX_T)
_KSIZE = _G * math.exp(-_T * _DENO)
_SIGMA = 0.3 * ((_KSIZE - 1) * 0.5 - 1) + 0.8
_TWO_SIGMA_SQ = 2.0 * _SIGMA**2
_NEG_INV_2SQ = -1.0 / _TWO_SIGMA_SQ
_TEMP = math.exp(-(_T * 2) / _MAX_T)
_LANES = 16


def _dist_body(x_ref, u_ref, dist_ref):
    d = u_ref.shape[1]
    xc = x_ref[:] - 0.5
    uc = u_ref[:] - 0.5
    dn = (((1,), (1,)), ((), ()))
    un = jax.lax.dot_general(
        jnp.ones((1, d), jnp.float32), uc * uc, dn,
        preferred_element_type=jnp.float32,
        precision=jax.lax.Precision.HIGHEST)
    dot = jax.lax.dot_general(
        xc, uc, dn, preferred_element_type=jnp.float32,
        precision=jax.lax.Precision.HIGHEST)
    xn = jnp.sum(xc * xc, axis=1, keepdims=True)
    dist_ref[:] = (xn - 2.0 * dot) + un


_GDN = lax.GatherDimensionNumbers(
    offset_dims=(), collapsed_slice_dims=(0,), start_index_map=(0,))
_GDN2 = lax.GatherDimensionNumbers(
    offset_dims=(), collapsed_slice_dims=(0, 1), start_index_map=(0, 1))


def _shuffle(v, idx):
    return lax.gather(
        v, idx[:, None], _GDN, (1,),
        mode=lax.GatherScatterMode.PROMISE_IN_BOUNDS)


def _hmin(v, lane):
    for sh in (8, 4, 2, 1):
        v = jnp.minimum(v, _shuffle(v, lane ^ sh))
    return v


def _sc_body(ns, spw, dist_hbm, stage_hbm, bmu_hbm, part_hbm,
             dvec, stage_vmem, psum_vmem, grid_vmem):
    n = dist_hbm.shape[1]
    nchunk = n // _LANES
    cid = lax.axis_index("c")
    sid = lax.axis_index("s")
    wid = cid * ns + sid
    pltpu.sync_copy(dist_hbm.at[pl.ds(wid * spw, spw)], dvec)
    lane = lax.iota(jnp.int32, _LANES)

    def amin_body(c, carry):
        col = c * _LANES + lane
        out = []
        for s in range(spw):
            minv, argv = carry[s]
            v = dvec[s, pl.ds(c * _LANES, _LANES)]
            m = v < minv
            out.append((jnp.where(m, v, minv), jnp.where(m, col, argv)))
        return tuple(out)

    init = tuple((dvec[s, pl.ds(0, _LANES)], lane) for s in range(spw))
    carry = lax.fori_loop(1, nchunk, amin_body, init)

    bmu_vec = jnp.zeros((_LANES,), jnp.int32)
    bys, bxs = [], []
    for s in range(spw):
        minv, argv = carry[s]
        mval = _hmin(minv, lane)
        bidx = _hmin(jnp.where(minv == mval, argv, jnp.int32(1 << 30)), lane)
        bmu_vec = jnp.where(lane == s, bidx, bmu_vec)
        bys.append(bidx >> 5)
        bxs.append(bidx & 31)

    def loss_body(c, accs):
        col = c * _LANES + lane
        uy = col >> 5
        ux = col & 31
        out = []
        for s in range(spw):
            v = dvec[s, pl.ds(c * _LANES, _LANES)]
            dy = uy - bys[s]
            dx = ux - bxs[s]
            pd = (dy * dy + dx * dx).astype(jnp.float32)
            g = jnp.exp(pd * _NEG_INV_2SQ)
            g = jnp.where(g < 0.001, 0.0, g)
            out.append(accs[s] + g * g * v)
        return tuple(out)

    zero = jnp.zeros((_LANES,), jnp.float32)
    accs = lax.fori_loop(0, nchunk, loss_body, (zero,) * spw)
    acc = accs[0]
    for s in range(1, spw):
        acc = acc + accs[s]

    stage_vmem[0, pl.ds(0, _LANES)] = bmu_vec
    stage_vmem[0, pl.ds(_LANES, _LANES)] = jax.lax.bitcast_convert_type(
        acc, jnp.int32)
    pltpu.sync_copy(stage_vmem.at[0], stage_hbm.at[wid])
    plsc.subcore_barrier()

    @pl.when(sid == 0)
    def _():
        pltpu.sync_copy(stage_hbm.at[pl.ds(cid * ns, ns)], grid_vmem)
        for q in range(spw):
            vals = jnp.zeros((_LANES,), jnp.int32)
            for j in range(_LANES // spw):
                rowv = grid_vmem[q * (_LANES // spw) + j, pl.ds(0, _LANES)]
                shuf = _shuffle(rowv, lane & (spw - 1))
                vals = jnp.where((lane >> 2) == j, shuf, vals)
            stage_vmem[q, pl.ds(0, _LANES)] = vals
        pltpu.sync_copy(stage_vmem.at[pl.ds(0, spw)],
                        bmu_hbm.at[pl.ds(cid * spw, spw)])
        psum = zero
        for r in range(ns):
            psum = psum + jax.lax.bitcast_convert_type(
                grid_vmem[r, pl.ds(_LANES, _LANES)], jnp.float32)
        psum_vmem[:] = psum
        pltpu.sync_copy(psum_vmem, part_hbm.at[cid])


def kernel(x, units):
    bsz = x.shape[0]
    d = units.shape[-1]
    n = units.shape[0] * units.shape[1]
    x2 = x.reshape(bsz, d)
    u2 = units.reshape(n, d)
    dist = pl.pallas_call(
        _dist_body,
        out_shape=jax.ShapeDtypeStruct((bsz, n), jnp.float32),
    )(x2, u2)

    info = plsc.get_sparse_core_info()
    nc, ns = info.num_cores, info.num_subcores
    spw = bsz // (nc * ns)
    mesh = plsc.VectorSubcoreMesh(core_axis_name="c", subcore_axis_name="s")
    sc = pl.kernel(
        functools.partial(_sc_body, ns, spw),
        mesh=mesh,
        out_type=(
            jax.ShapeDtypeStruct((nc * ns, 2 * _LANES), jnp.int32),
            jax.ShapeDtypeStruct((nc * spw, 2 * _LANES), jnp.int32),
            jax.ShapeDtypeStruct((nc, _LANES), jnp.float32),
        ),
        scratch_types=[
            pltpu.VMEM((spw, n), jnp.float32),
            pltpu.VMEM((spw, 2 * _LANES), jnp.int32),
            pltpu.VMEM((_LANES,), jnp.float32),
            pltpu.VMEM((ns, 2 * _LANES), jnp.int32),
        ],
    )
    _, bmu_parts, loss_parts = sc(dist)
    bmu = bmu_parts[:, :_LANES].reshape(bsz, 1)
    loss = jnp.sum(loss_parts) * (_TEMP * _TEMP / (bsz * n * d))
    return bmu, loss
